# baseline (speedup 1.0000x reference)
import functools

import jax
import jax.numpy as jnp
from jax import lax
from jax.experimental import pallas as pl
from jax.experimental.pallas import tpu as pltpu
from jax.experimental.pallas import tpu_sc as plsc

N = 10000
NP = 10240
E = 320000
PE = 327680
K = 128
ROWS = PE // K
ROWS_ALLOC = ROWS + 40
A_C0 = 160
B_C1 = (ROWS - 16 * A_C0) // 16
NC = 2
NS = 16
NW = NC * NS
RPT = ROWS // NW
RPT2 = RPT // 2
NPT = NP // NS
F = 128

_f32 = jnp.float32
_i32 = jnp.int32


_MESH = plsc.VectorSubcoreMesh(core_axis_name="c", subcore_axis_name="s")


@functools.partial(
    pl.kernel,
    out_type=jax.ShapeDtypeStruct((NC, NP), _f32),
    mesh=_MESH,
    scratch_types=[
        pltpu.VMEM((RPT, K), _i32),
        pltpu.VMEM((RPT, K), _f32),
        pltpu.VMEM((NPT,), _f32),
        pltpu.VMEM_SHARED((NP,), _f32),
    ],
)
def _deg_kernel(dst_hbm, ew_hbm, out_hbm, dstv, ewv, zv, acc):
    c = lax.axis_index("c")
    s = lax.axis_index("s")
    wid = c * NS + s

    def _zero(j, carry):
        zv[pl.ds(j * 16, 16)] = jnp.zeros((16,), _f32)
        return carry

    lax.fori_loop(0, NPT // 16, _zero, 0)
    pltpu.sync_copy(zv, acc.at[pl.ds(s * NPT, NPT)])
    plsc.subcore_barrier()

    pltpu.sync_copy(dst_hbm.at[pl.ds(wid * RPT, RPT)], dstv)
    pltpu.sync_copy(ew_hbm.at[pl.ds(wid * RPT, RPT)], ewv)

    def _chunk(k, carry):
        pltpu.sync_copy(ewv.at[k], acc.at[dstv.at[k]], add=True)
        return carry

    lax.fori_loop(0, RPT, _chunk, 0)
    plsc.subcore_barrier()
    pltpu.sync_copy(acc.at[pl.ds(s * NPT, NPT)], out_hbm.at[c, pl.ds(s * NPT, NPT)])


@functools.partial(
    pl.kernel,
    out_type=jax.ShapeDtypeStruct((NC, NP, F), _f32),
    mesh=_MESH,
    scratch_types=[
        pltpu.VMEM((RPT2, K), _i32),
        pltpu.VMEM((RPT2, K), _i32),
        pltpu.VMEM((RPT2, K), _f32),
        pltpu.VMEM((K, F), _f32),
        pltpu.VMEM((K, F), _f32),
        pltpu.VMEM_SHARED((NP, F), _f32),
        pltpu.SemaphoreType.DMA,
        pltpu.SemaphoreType.DMA,
        pltpu.SemaphoreType.DMA,
        pltpu.SemaphoreType.DMA,
    ],
)
def _scatter_kernel(y_hbm, src_hbm, dst_hbm, ew_hbm, out_hbm,
                    srcv, dstv, ewv, rows_a, rows_b, acc,
                    sem_a0, sem_a1, sem_b0, sem_b1):
    c = lax.axis_index("c")
    s = lax.axis_index("s")
    wid = c * NS + s

    def _zero(j, carry):
        for u in range(F // 16):
            rows_a[j, pl.ds(u * 16, 16)] = jnp.zeros((16,), _f32)
        return carry

    lax.fori_loop(0, K, _zero, 0)
    for t in range(NPT // K):
        pltpu.sync_copy(rows_a, acc.at[pl.ds(s * NPT + t * K, K)])
    plsc.subcore_barrier()

    def _scale(rows, k):
        def _scale16(g, inner):
            ew16 = ewv[k, pl.ds(g * 16, 16)]
            for j in range(16):
                sval = ew16.at[jnp.full((16,), j, _i32)].get(
                    mode="promise_in_bounds")
                e = g * 16 + j
                for u in range(F // 16):
                    sl = pl.ds(u * 16, 16)
                    rows[e, sl] = rows[e, sl] * sval
            return inner

        lax.fori_loop(0, K // 16, _scale16, 0)

    H = K // 2

    def _issue(k, buf, sems):
        for hh in range(2):
            pltpu.async_copy(y_hbm.at[srcv.at[k, pl.ds(hh * H, H)]],
                             buf.at[pl.ds(hh * H, H)], sems[hh])

    def _wait(k, buf, sems):
        for hh in range(2):
            pltpu.make_async_copy(y_hbm.at[srcv.at[k, pl.ds(hh * H, H)]],
                                  buf.at[pl.ds(hh * H, H)], sems[hh]).wait()

    def _chunk(k, cur, cur_sems, nxt, nxt_sems, prefetch):
        _wait(k, cur, cur_sems)
        if prefetch:
            _issue(k + 1, nxt, nxt_sems)
        _scale(cur, k)
        pltpu.sync_copy(cur, acc.at[dstv.at[k]], add=True)

    cnt = jnp.where(c == 0, A_C0, B_C1)
    row0 = jnp.where(c == 0, s * A_C0, NS * A_C0 + s * B_C1)
    nstages = (cnt + RPT2 - 1) // RPT2

    def _stage(h, carry):
        base = row0 + h * RPT2
        sc_ = jnp.minimum(cnt - h * RPT2, RPT2)
        pltpu.sync_copy(src_hbm.at[pl.ds(base, RPT2)], srcv)
        pltpu.sync_copy(dst_hbm.at[pl.ds(base, RPT2)], dstv)
        pltpu.sync_copy(ew_hbm.at[pl.ds(base, RPT2)], ewv)
        _issue(0, rows_a, (sem_a0, sem_a1))

        def _pair(m, inner):
            k0 = m * 2
            _chunk(k0, rows_a, (sem_a0, sem_a1), rows_b, (sem_b0, sem_b1), True)

            @pl.when(m < sc_ // 2 - 1)
            def _():
                _issue(k0 + 2, rows_a, (sem_a0, sem_a1))

            _chunk(k0 + 1, rows_b, (sem_b0, sem_b1), rows_a, (sem_a0, sem_a1),
                   False)
            return inner

        lax.fori_loop(0, sc_ // 2, _pair, 0)
        return carry

    lax.fori_loop(0, nstages, _stage, 0)
    plsc.subcore_barrier()
    for t in range(NPT // K):
        sl = pl.ds(s * NPT + t * K, K)
        pltpu.sync_copy(acc.at[sl], out_hbm.at[c, sl])


_BLK = 1024
_GRID = NP // _BLK


def _row_spec():
    return pl.BlockSpec((_BLK, F), lambda i: (i, 0))


def _full_spec():
    return pl.BlockSpec((F, F), lambda i: (0, 0))


def _vec_spec():
    return pl.BlockSpec((1, F), lambda i: (0, 0))


def _tc1_body(x_ref, w_ref, degb_ref, disb_ref, y_ref):
    disb = lax.rsqrt(degb_ref[...])
    disb_ref[...] = disb
    y_ref[...] = disb * jnp.dot(x_ref[...], w_ref[...],
                                preferred_element_type=_f32)


_tc1 = pl.pallas_call(
    _tc1_body,
    grid=(_GRID,),
    in_specs=[_row_spec(), _full_spec(), _row_spec()],
    out_specs=[_row_spec(), _row_spec()],
    out_shape=[jax.ShapeDtypeStruct((NP, F), _f32),
               jax.ShapeDtypeStruct((NP, F), _f32)],
)


def _tc2_body(z0_ref, z1_ref, y1_ref, disb_ref, b1_ref, w2_ref, y2_ref):
    disb = disb_ref[...]
    zsum = z0_ref[...] + z1_ref[...] + y1_ref[...]
    h = jnp.maximum(disb * zsum + b1_ref[...], 0.0)
    y2_ref[...] = disb * jnp.dot(h, w2_ref[...], preferred_element_type=_f32)


_tc2 = pl.pallas_call(
    _tc2_body,
    grid=(_GRID,),
    in_specs=[_row_spec(), _row_spec(), _row_spec(), _row_spec(),
              _vec_spec(), _full_spec()],
    out_specs=_row_spec(),
    out_shape=jax.ShapeDtypeStruct((NP, F), _f32),
)


def _tc3_body(z0_ref, z1_ref, y2_ref, disb_ref, b2_ref, out_ref):
    out_ref[...] = (disb_ref[...] * (z0_ref[...] + z1_ref[...] + y2_ref[...])
                    + b2_ref[...])


_tc3 = pl.pallas_call(
    _tc3_body,
    grid=(_GRID,),
    in_specs=[_row_spec(), _row_spec(), _row_spec(), _row_spec(), _vec_spec()],
    out_specs=_row_spec(),
    out_shape=jax.ShapeDtypeStruct((NP, F), _f32),
)


def kernel(x, edge_index, edge_attr, W1, b1, W2, b2):
    xp = jnp.pad(x, ((0, NP - N), (0, 0)))
    src2 = jnp.pad(edge_index[0], (0, ROWS_ALLOC * K - E)).reshape(ROWS_ALLOC, K)
    dst2 = jnp.pad(edge_index[1], (0, ROWS_ALLOC * K - E)).reshape(ROWS_ALLOC, K)
    ew2 = jnp.pad(edge_attr, (0, ROWS_ALLOC * K - E)).reshape(ROWS_ALLOC, K)
    b1r = b1.reshape(1, F)
    b2r = b2.reshape(1, F)

    degp = _deg_kernel(dst2, ew2)
    deg = degp[0] + degp[1] + 1.0
    degb = jnp.broadcast_to(deg[:, None], (NP, F))

    disb, y1 = _tc1(xp, W1, degb)
    z1 = _scatter_kernel(y1, src2, dst2, ew2)
    y2 = _tc2(z1[0], z1[1], y1, disb, b1r, W2)
    z2 = _scatter_kernel(y2, src2, dst2, ew2)
    outp = _tc3(z2[0], z2[1], y2, disb, b2r)
    return outp[:N]


# speedup vs baseline: 1.5764x; 1.5764x over previous
import functools

import jax
import jax.numpy as jnp
from jax import lax
from jax.experimental import pallas as pl
from jax.experimental.pallas import tpu as pltpu
from jax.experimental.pallas import tpu_sc as plsc

N = 10000
NP = 10240
E = 320000
PE = 327680
K = 128
ROWS = PE // K
ROWS_ALLOC = ROWS + 40
A_C0 = 152
B_C1 = (ROWS - 16 * A_C0) // 16
NC = 2
NS = 16
NW = NC * NS
RPT = ROWS // NW
RPT2 = RPT // 2
NPT = NP // NS
F = 128

_f32 = jnp.float32
_i32 = jnp.int32


_MESH = plsc.VectorSubcoreMesh(core_axis_name="c", subcore_axis_name="s")


@functools.partial(
    pl.kernel,
    out_type=jax.ShapeDtypeStruct((NC, NP), _f32),
    mesh=_MESH,
    scratch_types=[
        pltpu.VMEM((RPT, K), _i32),
        pltpu.VMEM((RPT, K), _f32),
        pltpu.VMEM((NPT,), _f32),
        pltpu.VMEM_SHARED((NP,), _f32),
    ],
)
def _deg_kernel(dst_hbm, ew_hbm, out_hbm, dstv, ewv, zv, acc):
    c = lax.axis_index("c")
    s = lax.axis_index("s")
    wid = c * NS + s

    def _zero(j, carry):
        zv[pl.ds(j * 16, 16)] = jnp.zeros((16,), _f32)
        return carry

    lax.fori_loop(0, NPT // 16, _zero, 0)
    pltpu.sync_copy(zv, acc.at[pl.ds(s * NPT, NPT)])
    plsc.subcore_barrier()

    pltpu.sync_copy(dst_hbm.at[pl.ds(wid * RPT, RPT)], dstv)
    pltpu.sync_copy(ew_hbm.at[pl.ds(wid * RPT, RPT)], ewv)

    def _chunk(k, carry):
        pltpu.sync_copy(ewv.at[k], acc.at[dstv.at[k]], add=True)
        return carry

    lax.fori_loop(0, RPT, _chunk, 0)
    plsc.subcore_barrier()
    pltpu.sync_copy(acc.at[pl.ds(s * NPT, NPT)], out_hbm.at[c, pl.ds(s * NPT, NPT)])


@functools.partial(
    pl.kernel,
    out_type=jax.ShapeDtypeStruct((NC, NP, F), _f32),
    mesh=_MESH,
    scratch_types=[
        pltpu.VMEM((RPT2, K), _i32),
        pltpu.VMEM((RPT2, K), _i32),
        pltpu.VMEM((RPT2, K), _f32),
        pltpu.VMEM((K, F), _f32),
        pltpu.VMEM((K, F), _f32),
        pltpu.VMEM_SHARED((NP, F), _f32),
        pltpu.SemaphoreType.DMA,
        pltpu.SemaphoreType.DMA,
        pltpu.SemaphoreType.DMA,
        pltpu.SemaphoreType.DMA,
    ],
)
def _scatter_kernel(y_hbm, src_hbm, dst_hbm, ew_hbm, out_hbm,
                    srcv, dstv, ewv, rows_a, rows_b, acc,
                    sem_a0, sem_a1, sem_b0, sem_b1):
    c = lax.axis_index("c")
    s = lax.axis_index("s")
    wid = c * NS + s

    def _zero(j, carry):
        for u in range(F // 16):
            rows_a[j, pl.ds(u * 16, 16)] = jnp.zeros((16,), _f32)
        return carry

    lax.fori_loop(0, K, _zero, 0)
    for t in range(NPT // K):
        pltpu.sync_copy(rows_a, acc.at[pl.ds(s * NPT + t * K, K)])
    plsc.subcore_barrier()

    def _scale(rows, k):
        def _scale16(g, inner):
            ew16 = ewv[k, pl.ds(g * 16, 16)]
            for j in range(16):
                sval = ew16.at[jnp.full((16,), j, _i32)].get(
                    mode="promise_in_bounds")
                e = g * 16 + j
                for u in range(F // 16):
                    sl = pl.ds(u * 16, 16)
                    rows[e, sl] = rows[e, sl] * sval
            return inner

        lax.fori_loop(0, K // 16, _scale16, 0)

    H = K // 2

    def _issue(k, buf, sems):
        for hh in range(2):
            pltpu.async_copy(y_hbm.at[srcv.at[k, pl.ds(hh * H, H)]],
                             buf.at[pl.ds(hh * H, H)], sems[hh])

    def _wait(k, buf, sems):
        for hh in range(2):
            pltpu.make_async_copy(y_hbm.at[srcv.at[k, pl.ds(hh * H, H)]],
                                  buf.at[pl.ds(hh * H, H)], sems[hh]).wait()

    def _chunk(k, cur, cur_sems, nxt, nxt_sems, prefetch):
        _wait(k, cur, cur_sems)
        if prefetch:
            _issue(k + 1, nxt, nxt_sems)
        _scale(cur, k)
        pltpu.sync_copy(cur, acc.at[dstv.at[k]], add=True)

    cnt = jnp.where(c == 0, A_C0, B_C1)
    row0 = jnp.where(c == 0, s * A_C0, NS * A_C0 + s * B_C1)
    nstages = (cnt + RPT2 - 1) // RPT2

    def _stage(h, carry):
        base = row0 + h * RPT2
        sc_ = jnp.minimum(cnt - h * RPT2, RPT2)
        pltpu.sync_copy(src_hbm.at[pl.ds(base, RPT2)], srcv)
        pltpu.sync_copy(dst_hbm.at[pl.ds(base, RPT2)], dstv)
        pltpu.sync_copy(ew_hbm.at[pl.ds(base, RPT2)], ewv)
        _issue(0, rows_a, (sem_a0, sem_a1))

        def _pair(m, inner):
            k0 = m * 2
            _chunk(k0, rows_a, (sem_a0, sem_a1), rows_b, (sem_b0, sem_b1), True)

            @pl.when(m < sc_ // 2 - 1)
            def _():
                _issue(k0 + 2, rows_a, (sem_a0, sem_a1))

            _chunk(k0 + 1, rows_b, (sem_b0, sem_b1), rows_a, (sem_a0, sem_a1),
                   False)
            return inner

        lax.fori_loop(0, sc_ // 2, _pair, 0)
        return carry

    lax.fori_loop(0, nstages, _stage, 0)
    plsc.subcore_barrier()
    for t in range(NPT // K):
        sl = pl.ds(s * NPT + t * K, K)
        pltpu.sync_copy(acc.at[sl], out_hbm.at[c, sl])


_BLK = 1024
_GRID = NP // _BLK


def _row_spec():
    return pl.BlockSpec((_BLK, F), lambda i: (i, 0))


def _full_spec():
    return pl.BlockSpec((F, F), lambda i: (0, 0))


def _vec_spec():
    return pl.BlockSpec((1, F), lambda i: (0, 0))


def _tc1_body(x_ref, w_ref, degb_ref, disb_ref, y_ref):
    disb = lax.rsqrt(degb_ref[...])
    disb_ref[...] = disb
    y_ref[...] = disb * jnp.dot(x_ref[...], w_ref[...],
                                preferred_element_type=_f32)


_tc1 = pl.pallas_call(
    _tc1_body,
    grid=(_GRID,),
    in_specs=[_row_spec(), _full_spec(), _row_spec()],
    out_specs=[_row_spec(), _row_spec()],
    out_shape=[jax.ShapeDtypeStruct((NP, F), _f32),
               jax.ShapeDtypeStruct((NP, F), _f32)],
)


def _tc2_body(z0_ref, z1_ref, y1_ref, disb_ref, b1_ref, w2_ref, y2_ref):
    disb = disb_ref[...]
    zsum = z0_ref[...] + z1_ref[...] + y1_ref[...]
    h = jnp.maximum(disb * zsum + b1_ref[...], 0.0)
    y2_ref[...] = disb * jnp.dot(h, w2_ref[...], preferred_element_type=_f32)


_tc2 = pl.pallas_call(
    _tc2_body,
    grid=(_GRID,),
    in_specs=[_row_spec(), _row_spec(), _row_spec(), _row_spec(),
              _vec_spec(), _full_spec()],
    out_specs=_row_spec(),
    out_shape=jax.ShapeDtypeStruct((NP, F), _f32),
)


def _tc3_body(z0_ref, z1_ref, y2_ref, disb_ref, b2_ref, out_ref):
    out_ref[...] = (disb_ref[...] * (z0_ref[...] + z1_ref[...] + y2_ref[...])
                    + b2_ref[...])


_tc3 = pl.pallas_call(
    _tc3_body,
    grid=(_GRID,),
    in_specs=[_row_spec(), _row_spec(), _row_spec(), _row_spec(), _vec_spec()],
    out_specs=_row_spec(),
    out_shape=jax.ShapeDtypeStruct((NP, F), _f32),
)


def kernel(x, edge_index, edge_attr, W1, b1, W2, b2):
    xp = jnp.pad(x, ((0, NP - N), (0, 0)))
    src2 = jnp.pad(edge_index[0], (0, ROWS_ALLOC * K - E)).reshape(ROWS_ALLOC, K)
    dst2 = jnp.pad(edge_index[1], (0, ROWS_ALLOC * K - E)).reshape(ROWS_ALLOC, K)
    ew2 = jnp.pad(edge_attr, (0, ROWS_ALLOC * K - E)).reshape(ROWS_ALLOC, K)
    b1r = b1.reshape(1, F)
    b2r = b2.reshape(1, F)

    degp = _deg_kernel(dst2, ew2)
    deg = degp[0] + degp[1] + 1.0
    degb = jnp.broadcast_to(deg[:, None], (NP, F))

    disb, y1 = _tc1(xp, W1, degb)
    z1 = _scatter_kernel(y1, src2, dst2, ew2)
    y2 = _tc2(z1[0], z1[1], y1, disb, b1r, W2)
    z2 = _scatter_kernel(y2, src2, dst2, ew2)
    outp = _tc3(z2[0], z2[1], y2, disb, b2r)
    return outp[:N]


# speedup vs baseline: 1.5767x; 1.0002x over previous
import functools

import jax
import jax.numpy as jnp
from jax import lax
from jax.experimental import pallas as pl
from jax.experimental.pallas import tpu as pltpu
from jax.experimental.pallas import tpu_sc as plsc

N = 10000
NP = 10240
E = 320000
PE = 327680
K = 128
ROWS = PE // K
ROWS_ALLOC = ROWS + 40
A_C0 = 152
B_C1 = (ROWS - 16 * A_C0) // 16
NC = 2
NS = 16
NW = NC * NS
RPT = ROWS // NW
RPT2 = RPT // 2
NPT = NP // NS
F = 128

_f32 = jnp.float32
_i32 = jnp.int32


_MESH = plsc.VectorSubcoreMesh(core_axis_name="c", subcore_axis_name="s")


@functools.partial(
    pl.kernel,
    out_type=jax.ShapeDtypeStruct((NC, NP), _f32),
    mesh=_MESH,
    scratch_types=[
        pltpu.VMEM((RPT, K), _i32),
        pltpu.VMEM((RPT, K), _f32),
        pltpu.VMEM((NPT,), _f32),
        pltpu.VMEM_SHARED((NP,), _f32),
    ],
)
def _deg_kernel(dst_hbm, ew_hbm, out_hbm, dstv, ewv, zv, acc):
    c = lax.axis_index("c")
    s = lax.axis_index("s")
    wid = c * NS + s

    def _zero(j, carry):
        zv[pl.ds(j * 16, 16)] = jnp.zeros((16,), _f32)
        return carry

    lax.fori_loop(0, NPT // 16, _zero, 0)
    pltpu.sync_copy(zv, acc.at[pl.ds(s * NPT, NPT)])
    plsc.subcore_barrier()

    pltpu.sync_copy(dst_hbm.at[pl.ds(wid * RPT, RPT)], dstv)
    pltpu.sync_copy(ew_hbm.at[pl.ds(wid * RPT, RPT)], ewv)

    def _chunk(k, carry):
        pltpu.sync_copy(ewv.at[k], acc.at[dstv.at[k]], add=True)
        return carry

    lax.fori_loop(0, RPT, _chunk, 0)
    plsc.subcore_barrier()
    pltpu.sync_copy(acc.at[pl.ds(s * NPT, NPT)], out_hbm.at[c, pl.ds(s * NPT, NPT)])


@functools.partial(
    pl.kernel,
    out_type=jax.ShapeDtypeStruct((NC, NP, F), _f32),
    mesh=_MESH,
    scratch_types=[
        pltpu.VMEM((RPT2, K), _i32),
        pltpu.VMEM((RPT2, K), _i32),
        pltpu.VMEM((RPT2, K), _f32),
        pltpu.VMEM((K, F), _f32),
        pltpu.VMEM((K, F), _f32),
        pltpu.VMEM_SHARED((NP, F), _f32),
        pltpu.SemaphoreType.DMA,
        pltpu.SemaphoreType.DMA,
        pltpu.SemaphoreType.DMA,
        pltpu.SemaphoreType.DMA,
    ],
)
def _scatter_kernel(y_hbm, src_hbm, dst_hbm, ew_hbm, out_hbm,
                    srcv, dstv, ewv, rows_a, rows_b, acc,
                    sem_a0, sem_a1, sem_b0, sem_b1):
    c = lax.axis_index("c")
    s = lax.axis_index("s")
    H = K // 2

    def _zero(j, carry):
        for u in range(F // 16):
            rows_a[j, pl.ds(u * 16, 16)] = jnp.zeros((16,), _f32)
        return carry

    lax.fori_loop(0, K, _zero, 0)
    for t in range(NPT // K):
        pltpu.sync_copy(rows_a, acc.at[pl.ds(s * NPT + t * K, K)])
    plsc.subcore_barrier()

    def _issue(k, buf, sems):
        for hh in range(2):
            pltpu.async_copy(y_hbm.at[srcv.at[k, pl.ds(hh * H, H)]],
                             buf.at[pl.ds(hh * H, H)], sems[hh])

    def _wait(k, buf, sems):
        for hh in range(2):
            pltpu.make_async_copy(y_hbm.at[srcv.at[k, pl.ds(hh * H, H)]],
                                  buf.at[pl.ds(hh * H, H)], sems[hh]).wait()

    def _scale(rows, k):
        def _scale16(g, inner):
            ew16 = ewv[k, pl.ds(g * 16, 16)]
            for j in range(16):
                sval = ew16.at[jnp.full((16,), j, _i32)].get(
                    mode="promise_in_bounds")
                e = g * 16 + j
                for u in range(F // 16):
                    sl = pl.ds(u * 16, 16)
                    rows[e, sl] = rows[e, sl] * sval
            return inner

        lax.fori_loop(0, K // 16, _scale16, 0)

    def _chunk(k, cur, cur_sems, nxt, nxt_sems, prefetch):
        _wait(k, cur, cur_sems)
        if prefetch:
            _issue(k + 1, nxt, nxt_sems)
        _scale(cur, k)
        pltpu.sync_copy(cur, acc.at[dstv.at[k]], add=True)

    cnt = jnp.where(c == 0, A_C0, B_C1)
    row0 = jnp.where(c == 0, s * A_C0, NS * A_C0 + s * B_C1)
    nstages = (cnt + RPT2 - 1) // RPT2

    def _stage(h, carry):
        base = row0 + h * RPT2
        sc_ = jnp.minimum(cnt - h * RPT2, RPT2)
        pltpu.sync_copy(src_hbm.at[pl.ds(base, RPT2)], srcv)
        pltpu.sync_copy(dst_hbm.at[pl.ds(base, RPT2)], dstv)
        pltpu.sync_copy(ew_hbm.at[pl.ds(base, RPT2)], ewv)
        _issue(0, rows_a, (sem_a0, sem_a1))

        def _pair(m, inner):
            k0 = m * 2
            _chunk(k0, rows_a, (sem_a0, sem_a1), rows_b, (sem_b0, sem_b1), True)

            @pl.when(m < sc_ // 2 - 1)
            def _():
                _issue(k0 + 2, rows_a, (sem_a0, sem_a1))

            _chunk(k0 + 1, rows_b, (sem_b0, sem_b1), rows_a, (sem_a0, sem_a1),
                   False)
            return inner

        lax.fori_loop(0, sc_ // 2, _pair, 0)
        return carry

    lax.fori_loop(0, nstages, _stage, 0)
    plsc.subcore_barrier()
    for t in range(NPT // K):
        sl = pl.ds(s * NPT + t * K, K)
        pltpu.sync_copy(acc.at[sl], out_hbm.at[c, sl])


_BLK = 1024
_GRID = NP // _BLK


def _row_spec():
    return pl.BlockSpec((_BLK, F), lambda i: (i, 0))


def _full_spec():
    return pl.BlockSpec((F, F), lambda i: (0, 0))


def _vec_spec():
    return pl.BlockSpec((1, F), lambda i: (0, 0))


def _tc1_body(x_ref, w_ref, degb_ref, disb_ref, y_ref):
    disb = lax.rsqrt(degb_ref[...])
    disb_ref[...] = disb
    y_ref[...] = disb * jnp.dot(x_ref[...], w_ref[...],
                                preferred_element_type=_f32)


_tc1 = pl.pallas_call(
    _tc1_body,
    grid=(_GRID,),
    in_specs=[_row_spec(), _full_spec(), _row_spec()],
    out_specs=[_row_spec(), _row_spec()],
    out_shape=[jax.ShapeDtypeStruct((NP, F), _f32),
               jax.ShapeDtypeStruct((NP, F), _f32)],
)


def _tc2_body(z0_ref, z1_ref, y1_ref, disb_ref, b1_ref, w2_ref, y2_ref):
    disb = disb_ref[...]
    zsum = z0_ref[...] + z1_ref[...] + y1_ref[...]
    h = jnp.maximum(disb * zsum + b1_ref[...], 0.0)
    y2_ref[...] = disb * jnp.dot(h, w2_ref[...], preferred_element_type=_f32)


_tc2 = pl.pallas_call(
    _tc2_body,
    grid=(_GRID,),
    in_specs=[_row_spec(), _row_spec(), _row_spec(), _row_spec(),
              _vec_spec(), _full_spec()],
    out_specs=_row_spec(),
    out_shape=jax.ShapeDtypeStruct((NP, F), _f32),
)


def _tc3_body(z0_ref, z1_ref, y2_ref, disb_ref, b2_ref, out_ref):
    out_ref[...] = (disb_ref[...] * (z0_ref[...] + z1_ref[...] + y2_ref[...])
                    + b2_ref[...])


_tc3 = pl.pallas_call(
    _tc3_body,
    grid=(_GRID,),
    in_specs=[_row_spec(), _row_spec(), _row_spec(), _row_spec(), _vec_spec()],
    out_specs=_row_spec(),
    out_shape=jax.ShapeDtypeStruct((NP, F), _f32),
)


def kernel(x, edge_index, edge_attr, W1, b1, W2, b2):
    xp = jnp.pad(x, ((0, NP - N), (0, 0)))
    src2 = jnp.pad(edge_index[0], (0, ROWS_ALLOC * K - E)).reshape(ROWS_ALLOC, K)
    dst2 = jnp.pad(edge_index[1], (0, ROWS_ALLOC * K - E)).reshape(ROWS_ALLOC, K)
    ew2 = jnp.pad(edge_attr, (0, ROWS_ALLOC * K - E)).reshape(ROWS_ALLOC, K)
    b1r = b1.reshape(1, F)
    b2r = b2.reshape(1, F)

    degp = _deg_kernel(dst2, ew2)
    deg = degp[0] + degp[1] + 1.0
    degb = jnp.broadcast_to(deg[:, None], (NP, F))

    disb, y1 = _tc1(xp, W1, degb)
    z1 = _scatter_kernel(y1, src2, dst2, ew2)
    y2 = _tc2(z1[0], z1[1], y1, disb, b1r, W2)
    z2 = _scatter_kernel(y2, src2, dst2, ew2)
    outp = _tc3(z2[0], z2[1], y2, disb, b2r)
    return outp[:N]
